# Initial kernel scaffold; baseline (speedup 1.0000x reference)
import jax
import jax.numpy as jnp
from jax import lax
from jax.experimental import pallas as pl
from jax.experimental.pallas import tpu as pltpu
from jax.experimental.pallas import tpu_sc as plsc

N_NODES = 10000
E = 160000
D = 256
DH = 128
GENE_NUM = 2000
NC = 2
NS = 16
ET = E // NS
CH = 80
NCHUNK = ET // CH
RT = N_NODES // NS
APAD = 2048


def _sc_kernel(h2, src2, dst3, ew2, cell, alpha_p,
               zsum, zcnt, outsum, outcnt,
               src_v, dstix_v, ew_v, cell_v, alpha_v, s_v, gidx_v,
               rows_v, ones_v, acc_sp, cnt_sp, gsem):
    c = lax.axis_index("c")
    s = lax.axis_index("s")

    pltpu.sync_copy(src2.at[s], src_v)
    pltpu.sync_copy(dst3.at[s], dstix_v)
    pltpu.sync_copy(ew2.at[s], ew_v)
    pltpu.sync_copy(cell, cell_v)
    pltpu.sync_copy(alpha_p, alpha_v)

    r0 = s * RT
    pltpu.sync_copy(zsum, acc_sp.at[pl.ds(r0, RT)])
    pltpu.sync_copy(zcnt, cnt_sp.at[pl.ds(r0, RT)])

    def set_ones(r, carry):
        ones_v[r] = jnp.full((16,), 1.0, jnp.float32)
        return carry
    lax.fori_loop(0, CH, set_ones, 0)

    def p0(j, carry):
        for k in range(CH // 16):
            off = j * CH + k * 16
            src16 = src_v[pl.ds(off, 16)]
            dst16 = dstix_v[j, pl.ds(k * 16, 16)]
            sid = plsc.load_gather(cell_v, [src16])
            did = plsc.load_gather(cell_v, [dst16])
            idx = jnp.full((16,), GENE_NUM + 1, jnp.int32)
            idx = jnp.where((sid >= 0) & (did < 0), sid, idx)
            idx = jnp.where((did >= 0) & (sid < 0), did, idx)
            idx = jnp.where((did >= 0) & (sid >= 0),
                            jnp.full((16,), GENE_NUM, jnp.int32), idx)
            a16 = plsc.load_gather(alpha_v, [idx])
            s_v[pl.ds(off, 16)] = a16 * ew_v[pl.ds(off, 16)]
            gidx_v[j, pl.ds(k * 16, 16)] = src16 * 2 + c
        return carry
    lax.fori_loop(0, NCHUNK, p0, 0)

    plsc.subcore_barrier()

    def p1(j, carry):
        pltpu.async_copy(h2.at[gidx_v.at[j]], rows_v, gsem).wait()

        def scale_row(r, carry2):
            sbc = plsc.load_gather(
                s_v, [jnp.full((16,), j * CH + r, jnp.int32)])
            for q in range(DH // 16):
                rows_v[r, pl.ds(q * 16, 16)] = (
                    rows_v[r, pl.ds(q * 16, 16)] * sbc)
            return carry2
        lax.fori_loop(0, CH, scale_row, 0)

        pltpu.sync_copy(rows_v, acc_sp.at[dstix_v.at[j]], add=True)
        pltpu.sync_copy(ones_v, cnt_sp.at[dstix_v.at[j]], add=True)
        return carry
    lax.fori_loop(0, NCHUNK, p1, 0)

    plsc.subcore_barrier()
    pltpu.sync_copy(acc_sp.at[pl.ds(r0, RT)], outsum.at[c, pl.ds(r0, RT)])
    pltpu.sync_copy(cnt_sp.at[pl.ds(r0, RT)], outcnt.at[c, pl.ds(r0, RT)])


def _sc_aggregate(h2, src2, dst3, ew2, cell, alpha_p, zsum, zcnt):
    mesh = plsc.VectorSubcoreMesh(core_axis_name="c", subcore_axis_name="s")
    return pl.kernel(
        _sc_kernel,
        out_type=[
            jax.ShapeDtypeStruct((NC, N_NODES, DH), jnp.float32),
            jax.ShapeDtypeStruct((NC, N_NODES, 16), jnp.float32),
        ],
        mesh=mesh,
        scratch_types=[
            pltpu.VMEM((ET,), jnp.int32),
            pltpu.VMEM((NCHUNK, CH), jnp.int32),
            pltpu.VMEM((ET,), jnp.float32),
            pltpu.VMEM((N_NODES,), jnp.int32),
            pltpu.VMEM((APAD,), jnp.float32),
            pltpu.VMEM((ET,), jnp.float32),
            pltpu.VMEM((NCHUNK, CH), jnp.int32),
            pltpu.VMEM((CH, DH), jnp.float32),
            pltpu.VMEM((CH, 16), jnp.float32),
            pltpu.VMEM_SHARED((N_NODES, DH), jnp.float32),
            pltpu.VMEM_SHARED((N_NODES, 16), jnp.float32),
            pltpu.SemaphoreType.DMA,
        ],
    )(h2, src2, dst3, ew2, cell, alpha_p, zsum, zcnt)


BR = 2000


def _tc_kernel(acc_ref, cnt_ref, w_ref, b_ref, g_ref, be_ref, o_ref):
    nb = jnp.concatenate([acc_ref[0], acc_ref[1]], axis=1)
    cntcol = cnt_ref[0][:, 0:1]
    neigh = jnp.where(cntcol > 0.0, nb / jnp.maximum(cntcol, 1.0), 0.0)
    z = lax.dot_general(neigh, w_ref[...], (((1,), (1,)), ((), ())),
                        preferred_element_type=jnp.float32)
    z = z + b_ref[...]
    z = jnp.maximum(z, 0.0)
    mu = jnp.mean(z, axis=1, keepdims=True)
    var = jnp.mean((z - mu) ** 2, axis=1, keepdims=True)
    o_ref[...] = (z - mu) / jnp.sqrt(var + 1e-5) * g_ref[...] + be_ref[...]


def _tc_tail(outsum, outcnt, W, b2, g2, be2):
    grid = (N_NODES // BR,)
    return pl.pallas_call(
        _tc_kernel,
        grid=grid,
        in_specs=[
            pl.BlockSpec((NC, BR, DH), lambda i: (0, i, 0)),
            pl.BlockSpec((1, BR, 16), lambda i: (0, i, 0)),
            pl.BlockSpec((D, D), lambda i: (0, 0)),
            pl.BlockSpec((1, D), lambda i: (0, 0)),
            pl.BlockSpec((1, D), lambda i: (0, 0)),
            pl.BlockSpec((1, D), lambda i: (0, 0)),
        ],
        out_specs=pl.BlockSpec((BR, D), lambda i: (i, 0)),
        out_shape=jax.ShapeDtypeStruct((N_NODES, D), jnp.float32),
    )(outsum, outcnt, W, b2, g2, be2)


@jax.jit
def kernel(h, edge_index, cell_id, edge_weight, alpha, W, b, gamma, beta):
    h2 = h.reshape(2 * N_NODES, DH)
    src2 = edge_index[0].reshape(NS, ET)
    dst3 = edge_index[1].reshape(NS, NCHUNK, CH)
    ew2 = edge_weight.reshape(NS, ET)
    alpha_p = jnp.zeros((APAD,), jnp.float32).at[: alpha.shape[0]].set(alpha)
    zsum = jnp.zeros((RT, DH), jnp.float32)
    zcnt = jnp.zeros((RT, 16), jnp.float32)

    outsum, outcnt = _sc_aggregate(h2, src2, dst3, ew2, cell_id, alpha_p,
                                   zsum, zcnt)

    b2 = b.reshape(1, D)
    g2 = gamma.reshape(1, D)
    be2 = beta.reshape(1, D)
    return _tc_tail(outsum, outcnt, W, b2, g2, be2)


# speedup vs baseline: 7.6922x; 7.6922x over previous
import jax
import jax.numpy as jnp
from jax import lax
from jax.experimental import pallas as pl
from jax.experimental.pallas import tpu as pltpu
from jax.experimental.pallas import tpu_sc as plsc

N_NODES = 10000
E = 160000
D = 256
DQ = 64
GENE_NUM = 2000
NC = 2
NS = 16
ET = E // NS
CH = 80
NCHUNK = ET // CH
N_PAD = 10240
RT = N_PAD // NS
APAD = 2048


def _sc_kernel(h4, src2, dst3, ew2, cell, alpha_p,
               zsum, zcnt, outsum, outcnt,
               src_v, dstix_v, ew_v, cell_v, alpha_v, s_v, gidx0_v, gidx1_v,
               rows_v, ones_v, acc_sp, cnt_sp, gsem):
    c = lax.axis_index("c")
    s = lax.axis_index("s")

    pltpu.sync_copy(src2.at[s], src_v)
    pltpu.sync_copy(dst3.at[s], dstix_v)
    pltpu.sync_copy(ew2.at[s], ew_v)
    pltpu.sync_copy(cell, cell_v)
    pltpu.sync_copy(alpha_p, alpha_v)

    r0 = s * RT

    def set_ones(r, carry):
        ones_v[r] = jnp.full((16,), 1.0, jnp.float32)
        return carry
    lax.fori_loop(0, CH, set_ones, 0)

    def p0(j, carry):
        for k in range(CH // 16):
            off = j * CH + k * 16
            src16 = src_v[pl.ds(off, 16)]
            dst16 = dstix_v[j, pl.ds(k * 16, 16)]
            sid = plsc.load_gather(cell_v, [src16])
            did = plsc.load_gather(cell_v, [dst16])
            idx = jnp.full((16,), GENE_NUM + 1, jnp.int32)
            idx = jnp.where((sid >= 0) & (did < 0), sid, idx)
            idx = jnp.where((did >= 0) & (sid < 0), did, idx)
            idx = jnp.where((did >= 0) & (sid >= 0),
                            jnp.full((16,), GENE_NUM, jnp.int32), idx)
            a16 = plsc.load_gather(alpha_v, [idx])
            s_v[pl.ds(off, 16)] = a16 * ew_v[pl.ds(off, 16)]
            g0 = src16 * 4 + c * 2
            gidx0_v[j, pl.ds(k * 16, 16)] = g0
            gidx1_v[j, pl.ds(k * 16, 16)] = g0 + 1
        return carry
    lax.fori_loop(0, NCHUNK, p0, 0)

    for p in range(2):
        pltpu.sync_copy(zsum, acc_sp.at[pl.ds(r0, RT)])
        if p == 0:
            pltpu.sync_copy(zcnt, cnt_sp.at[pl.ds(r0, RT)])
        plsc.subcore_barrier()

        gix = gidx0_v if p == 0 else gidx1_v

        def p1(j, carry):
            pltpu.async_copy(h4.at[gix.at[j]], rows_v, gsem).wait()

            def scale_row(r, carry2):
                sbc = plsc.load_gather(
                    s_v, [jnp.full((16,), j * CH + r, jnp.int32)])
                for q in range(DQ // 16):
                    rows_v[r, pl.ds(q * 16, 16)] = (
                        rows_v[r, pl.ds(q * 16, 16)] * sbc)
                return carry2
            lax.fori_loop(0, CH, scale_row, 0)

            pltpu.sync_copy(rows_v, acc_sp.at[dstix_v.at[j]], add=True)
            if p == 0:
                pltpu.sync_copy(ones_v, cnt_sp.at[dstix_v.at[j]], add=True)
            return carry
        lax.fori_loop(0, NCHUNK, p1, 0)

        plsc.subcore_barrier()
        pltpu.sync_copy(acc_sp.at[pl.ds(r0, RT)],
                        outsum.at[c, p, pl.ds(r0, RT)])
        if p == 0:
            pltpu.sync_copy(cnt_sp.at[pl.ds(r0, RT)],
                            outcnt.at[c, pl.ds(r0, RT)])


def _sc_aggregate(h4, src2, dst3, ew2, cell, alpha_p, zsum, zcnt):
    mesh = plsc.VectorSubcoreMesh(core_axis_name="c", subcore_axis_name="s")
    return pl.kernel(
        _sc_kernel,
        out_type=[
            jax.ShapeDtypeStruct((NC, 2, N_PAD, DQ), jnp.float32),
            jax.ShapeDtypeStruct((NC, N_PAD, 16), jnp.float32),
        ],
        mesh=mesh,
        compiler_params=pltpu.CompilerParams(needs_layout_passes=False, use_tc_tiling_on_sc=False),
        scratch_types=[
            pltpu.VMEM((ET,), jnp.int32),
            pltpu.VMEM((NCHUNK, CH), jnp.int32),
            pltpu.VMEM((ET,), jnp.float32),
            pltpu.VMEM((N_NODES,), jnp.int32),
            pltpu.VMEM((APAD,), jnp.float32),
            pltpu.VMEM((ET,), jnp.float32),
            pltpu.VMEM((NCHUNK, CH), jnp.int32),
            pltpu.VMEM((NCHUNK, CH), jnp.int32),
            pltpu.VMEM((CH, DQ), jnp.float32),
            pltpu.VMEM((CH, 16), jnp.float32),
            pltpu.VMEM_SHARED((N_PAD, DQ), jnp.float32),
            pltpu.VMEM_SHARED((N_PAD, 16), jnp.float32),
            pltpu.SemaphoreType.DMA,
        ],
    )(h4, src2, dst3, ew2, cell, alpha_p, zsum, zcnt)


BR = 2000


def _tc_kernel(acc_ref, cnt_ref, w_ref, b_ref, g_ref, be_ref, o_ref):
    nb = jnp.concatenate(
        [acc_ref[0, 0], acc_ref[0, 1], acc_ref[1, 0], acc_ref[1, 1]],
        axis=1)
    cntcol = cnt_ref[0][:, 0:1]
    neigh = jnp.where(cntcol > 0.0, nb / jnp.maximum(cntcol, 1.0), 0.0)
    z = lax.dot_general(neigh, w_ref[...], (((1,), (1,)), ((), ())),
                        preferred_element_type=jnp.float32)
    z = z + b_ref[...]
    z = jnp.maximum(z, 0.0)
    mu = jnp.mean(z, axis=1, keepdims=True)
    var = jnp.mean((z - mu) ** 2, axis=1, keepdims=True)
    o_ref[...] = (z - mu) / jnp.sqrt(var + 1e-5) * g_ref[...] + be_ref[...]


def _tc_tail(outsum, outcnt, W, b2, g2, be2):
    grid = (N_NODES // BR,)
    return pl.pallas_call(
        _tc_kernel,
        grid=grid,
        in_specs=[
            pl.BlockSpec((NC, 2, BR, DQ), lambda i: (0, 0, i, 0)),
            pl.BlockSpec((1, BR, 16), lambda i: (0, i, 0)),
            pl.BlockSpec((D, D), lambda i: (0, 0)),
            pl.BlockSpec((1, D), lambda i: (0, 0)),
            pl.BlockSpec((1, D), lambda i: (0, 0)),
            pl.BlockSpec((1, D), lambda i: (0, 0)),
        ],
        out_specs=pl.BlockSpec((BR, D), lambda i: (i, 0)),
        out_shape=jax.ShapeDtypeStruct((N_NODES, D), jnp.float32),
    )(outsum, outcnt, W, b2, g2, be2)


@jax.jit
def kernel(h, edge_index, cell_id, edge_weight, alpha, W, b, gamma, beta):
    h4 = h.reshape(4 * N_NODES, DQ)
    src2 = edge_index[0].reshape(NS, ET)
    dst3 = edge_index[1].reshape(NS, NCHUNK, CH)
    ew2 = edge_weight.reshape(NS, ET)
    alpha_p = jnp.zeros((APAD,), jnp.float32).at[: alpha.shape[0]].set(alpha)
    zsum = jnp.zeros((RT, DQ), jnp.float32)
    zcnt = jnp.zeros((RT, 16), jnp.float32)

    outsum, outcnt = _sc_aggregate(h4, src2, dst3, ew2, cell_id, alpha_p,
                                   zsum, zcnt)

    b2 = b.reshape(1, D)
    g2 = gamma.reshape(1, D)
    be2 = beta.reshape(1, D)
    return _tc_tail(outsum, outcnt, W, b2, g2, be2)


# speedup vs baseline: 12.7347x; 1.6555x over previous
import jax
import jax.numpy as jnp
from jax import lax
from jax.experimental import pallas as pl
from jax.experimental.pallas import tpu as pltpu
from jax.experimental.pallas import tpu_sc as plsc

N_NODES = 10000
E = 160000
D = 256
DQ = 64
GENE_NUM = 2000
NC = 2
NS = 16
ET = E // NS
CH = 80
NCHUNK = ET // CH
N_PAD = 10240
RT = N_PAD // NS
APAD = 2048


def _sc_kernel(h4, src2, dst3, ew2, cell, alpha_p,
               zsum, zcnt, ones_h, outsum, outcnt,
               src_v, dstix_v, ew_v, cell_v, alpha_v, s_v, gidx0_v, gidx1_v,
               rows_a, rows_b, ones_v, acc_sp, cnt_sp, gsemA, gsemB):
    c = lax.axis_index("c")
    s = lax.axis_index("s")

    pltpu.sync_copy(src2.at[s], src_v)
    pltpu.sync_copy(dst3.at[s], dstix_v)
    pltpu.sync_copy(ew2.at[s], ew_v)
    pltpu.sync_copy(cell, cell_v)
    pltpu.sync_copy(alpha_p, alpha_v)

    r0 = s * RT

    pltpu.sync_copy(ones_h, ones_v)

    def p0(j, carry):
        for k in range(CH // 16):
            off = j * CH + k * 16
            src16 = src_v[pl.ds(off, 16)]
            dst16 = dstix_v[j, pl.ds(k * 16, 16)]
            sid = plsc.load_gather(cell_v, [src16])
            did = plsc.load_gather(cell_v, [dst16])
            idx = jnp.full((16,), GENE_NUM + 1, jnp.int32)
            idx = jnp.where((sid >= 0) & (did < 0), sid, idx)
            idx = jnp.where((did >= 0) & (sid < 0), did, idx)
            idx = jnp.where((did >= 0) & (sid >= 0),
                            jnp.full((16,), GENE_NUM, jnp.int32), idx)
            a16 = plsc.load_gather(alpha_v, [idx])
            s_v[pl.ds(off, 16)] = a16 * ew_v[pl.ds(off, 16)]
            g0 = src16 * 4 + c * 2
            gidx0_v[j, pl.ds(k * 16, 16)] = g0
            gidx1_v[j, pl.ds(k * 16, 16)] = g0 + 1
        return carry
    lax.fori_loop(0, NCHUNK, p0, 0)

    for p in range(2):
        pltpu.sync_copy(zsum, acc_sp.at[pl.ds(r0, RT)])
        if p == 0:
            pltpu.sync_copy(zcnt, cnt_sp.at[pl.ds(r0, RT)])
        plsc.subcore_barrier()

        gix = gidx0_v if p == 0 else gidx1_v

        def scale_buf(buf, j):
            base = j * CH

            def srow(r4, carry2):
                for u in range(4):
                    r = r4 * 4 + u
                    sbc = plsc.load_gather(
                        s_v, [jnp.full((16,), base + r, jnp.int32)])
                    for q in range(DQ // 16):
                        buf[r, pl.ds(q * 16, 16)] = (
                            buf[r, pl.ds(q * 16, 16)] * sbc)
                return carry2
            lax.fori_loop(0, CH // 4, srow, 0)

        def consume(buf, j):
            scale_buf(buf, j)
            pltpu.sync_copy(buf, acc_sp.at[dstix_v.at[j]], add=True)
            if p == 0:
                pltpu.sync_copy(ones_v, cnt_sp.at[dstix_v.at[j]], add=True)

        pltpu.async_copy(h4.at[gix.at[0]], rows_a, gsemA)

        def p1(g, carry):
            ja = 2 * g
            jb = 2 * g + 1
            pltpu.make_async_copy(h4.at[gix.at[ja]], rows_a, gsemA).wait()
            pltpu.async_copy(h4.at[gix.at[jb]], rows_b, gsemB)
            consume(rows_a, ja)
            pltpu.make_async_copy(h4.at[gix.at[jb]], rows_b, gsemB).wait()
            jn = jnp.minimum(ja + 2, NCHUNK - 1)
            pltpu.async_copy(h4.at[gix.at[jn]], rows_a, gsemA)
            consume(rows_b, jb)
            return carry
        lax.fori_loop(0, (NCHUNK - 1) // 2, p1, 0)

        jt = NCHUNK - 1
        pltpu.make_async_copy(h4.at[gix.at[jt]], rows_a, gsemA).wait()
        consume(rows_a, jt)

        plsc.subcore_barrier()
        pltpu.sync_copy(acc_sp.at[pl.ds(r0, RT)],
                        outsum.at[c, p, pl.ds(r0, RT)])
        if p == 0:
            pltpu.sync_copy(cnt_sp.at[pl.ds(r0, RT)],
                            outcnt.at[c, pl.ds(r0, RT)])


def _sc_aggregate(h4, src2, dst3, ew2, cell, alpha_p, zsum, zcnt, ones_h):
    mesh = plsc.VectorSubcoreMesh(core_axis_name="c", subcore_axis_name="s")
    return pl.kernel(
        _sc_kernel,
        out_type=[
            jax.ShapeDtypeStruct((NC, 2, N_PAD, DQ), jnp.float32),
            jax.ShapeDtypeStruct((NC, N_PAD, 8), jnp.float32),
        ],
        mesh=mesh,
        compiler_params=pltpu.CompilerParams(needs_layout_passes=False, use_tc_tiling_on_sc=False),
        scratch_types=[
            pltpu.VMEM((ET,), jnp.int32),
            pltpu.VMEM((NCHUNK, CH), jnp.int32),
            pltpu.VMEM((ET,), jnp.float32),
            pltpu.VMEM((N_NODES,), jnp.int32),
            pltpu.VMEM((APAD,), jnp.float32),
            pltpu.VMEM((ET,), jnp.float32),
            pltpu.VMEM((NCHUNK, CH), jnp.int32),
            pltpu.VMEM((NCHUNK, CH), jnp.int32),
            pltpu.VMEM((CH, DQ), jnp.float32),
            pltpu.VMEM((CH, DQ), jnp.float32),
            pltpu.VMEM((CH, 8), jnp.float32),
            pltpu.VMEM_SHARED((N_PAD, DQ), jnp.float32),
            pltpu.VMEM_SHARED((N_PAD, 8), jnp.float32),
            pltpu.SemaphoreType.DMA,
            pltpu.SemaphoreType.DMA,
        ],
    )(h4, src2, dst3, ew2, cell, alpha_p, zsum, zcnt, ones_h)


BR = 2000


def _tc_kernel(acc_ref, cnt_ref, w_ref, b_ref, g_ref, be_ref, o_ref):
    nb = jnp.concatenate(
        [acc_ref[0, 0], acc_ref[0, 1], acc_ref[1, 0], acc_ref[1, 1]],
        axis=1)
    cntcol = cnt_ref[0][:, 0:1]
    neigh = jnp.where(cntcol > 0.0, nb / jnp.maximum(cntcol, 1.0), 0.0)
    z = lax.dot_general(neigh, w_ref[...], (((1,), (1,)), ((), ())),
                        preferred_element_type=jnp.float32)
    z = z + b_ref[...]
    z = jnp.maximum(z, 0.0)
    mu = jnp.mean(z, axis=1, keepdims=True)
    var = jnp.mean((z - mu) ** 2, axis=1, keepdims=True)
    o_ref[...] = (z - mu) / jnp.sqrt(var + 1e-5) * g_ref[...] + be_ref[...]


def _tc_tail(outsum, outcnt, W, b2, g2, be2):
    grid = (N_NODES // BR,)
    return pl.pallas_call(
        _tc_kernel,
        grid=grid,
        in_specs=[
            pl.BlockSpec((NC, 2, BR, DQ), lambda i: (0, 0, i, 0)),
            pl.BlockSpec((1, BR, 8), lambda i: (0, i, 0)),
            pl.BlockSpec((D, D), lambda i: (0, 0)),
            pl.BlockSpec((1, D), lambda i: (0, 0)),
            pl.BlockSpec((1, D), lambda i: (0, 0)),
            pl.BlockSpec((1, D), lambda i: (0, 0)),
        ],
        out_specs=pl.BlockSpec((BR, D), lambda i: (i, 0)),
        out_shape=jax.ShapeDtypeStruct((N_NODES, D), jnp.float32),
    )(outsum, outcnt, W, b2, g2, be2)


@jax.jit
def kernel(h, edge_index, cell_id, edge_weight, alpha, W, b, gamma, beta):
    h4 = h.reshape(4 * N_NODES, DQ)
    src2 = edge_index[0].reshape(NS, ET)
    dst3 = edge_index[1].reshape(NS, NCHUNK, CH)
    ew2 = edge_weight.reshape(NS, ET)
    alpha_p = jnp.zeros((APAD,), jnp.float32).at[: alpha.shape[0]].set(alpha)
    zsum = jnp.zeros((RT, DQ), jnp.float32)
    zcnt = jnp.zeros((RT, 8), jnp.float32)
    ones_h = jnp.ones((CH, 8), jnp.float32)

    outsum, outcnt = _sc_aggregate(h4, src2, dst3, ew2, cell_id, alpha_p,
                                   zsum, zcnt, ones_h)

    b2 = b.reshape(1, D)
    g2 = gamma.reshape(1, D)
    be2 = beta.reshape(1, D)
    return _tc_tail(outsum, outcnt, W, b2, g2, be2)
